# Initial kernel scaffold; baseline (speedup 1.0000x reference)
import jax
import jax.numpy as jnp
from jax.experimental import pallas as pl


def kernel(x, edge_index, edge_weight, edge_attr, W1, b1, W2, b2, W3, b3, W4, b4):
    raise NotImplementedError("write your pallas kernel here")


# speedup vs baseline: 2.7810x; 2.7810x over previous
import functools

import jax
import jax.numpy as jnp
from jax import lax
from jax.experimental import pallas as pl
from jax.experimental.pallas import tpu as pltpu
from jax.experimental.pallas import tpu_sc as plsc

N, E, F, EF = 10000, 320000, 128, 4
CUTOFF = 5.0
NC, NS = 2, 16
NW = NC * NS
EPW = E // NW
C = 80
NCH = EPW // C
NPT = 624
NTAIL = N - NS * NPT

_mesh = plsc.VectorSubcoreMesh(core_axis_name="c", subcore_axis_name="s")


def _f32(*shape):
    return jax.ShapeDtypeStruct(shape, jnp.float32)


@functools.partial(
    pl.kernel, mesh=_mesh,
    out_type=(_f32(E, F), _f32(E, F)),
    scratch_types=[
        pltpu.VMEM((C,), jnp.int32), pltpu.VMEM((C,), jnp.int32),
        pltpu.VMEM((C, F), jnp.float32), pltpu.VMEM((C, F), jnp.float32),
        pltpu.SemaphoreType.DMA, pltpu.SemaphoreType.DMA,
    ],
)
def _sc_gather(ta, tb, row, col, ga, gb, idxa, idxb, bufa, bufb, sema, semb):
    wid = lax.axis_index("s") * NC + lax.axis_index("c")
    base0 = wid * EPW

    def body(c, carry):
        b = base0 + c * C
        pltpu.sync_copy(row.at[pl.ds(b, C)], idxa)
        pltpu.sync_copy(col.at[pl.ds(b, C)], idxb)
        cpa = pltpu.async_copy(ta.at[idxa], bufa, sema)
        cpb = pltpu.async_copy(tb.at[idxb], bufb, semb)
        cpa.wait()
        cpb.wait()
        pltpu.sync_copy(bufa, ga.at[pl.ds(b, C)])
        pltpu.sync_copy(bufb, gb.at[pl.ds(b, C)])
        return carry

    lax.fori_loop(0, NCH, body, 0)


@functools.partial(
    pl.kernel, mesh=_mesh,
    out_type=_f32(NC * N, F),
    scratch_types=[
        pltpu.VMEM_SHARED((N, F), jnp.float32),
        pltpu.VMEM((C,), jnp.int32),
        pltpu.VMEM((C, F), jnp.float32),
    ],
)
def _sc_scatter(mij, row, zz, aggp, acc, idxv, buf):
    cid = lax.axis_index("c")
    sid = lax.axis_index("s")
    pltpu.sync_copy(zz.at[pl.ds(sid * NPT, NPT)], acc.at[pl.ds(sid * NPT, NPT)])

    @pl.when(sid == 0)
    def _():
        pltpu.sync_copy(zz.at[pl.ds(NS * NPT, NTAIL)],
                        acc.at[pl.ds(NS * NPT, NTAIL)])

    plsc.subcore_barrier()
    base0 = (sid * NC + cid) * EPW

    def body(c, carry):
        b = base0 + c * C
        pltpu.sync_copy(row.at[pl.ds(b, C)], idxv)
        pltpu.sync_copy(mij.at[pl.ds(b, C)], buf)
        pltpu.sync_copy(buf, acc.at[idxv], add=True)
        return carry

    lax.fori_loop(0, NCH, body, 0)
    plsc.subcore_barrier()
    pltpu.sync_copy(acc.at[pl.ds(sid * NPT, NPT)],
                    aggp.at[pl.ds(cid * N + sid * NPT, NPT)])

    @pl.when(sid == 0)
    def _():
        pltpu.sync_copy(acc.at[pl.ds(NS * NPT, NTAIL)],
                        aggp.at[pl.ds(cid * N + NS * NPT, NTAIL)])


BN = 1000


def _prep_body(x, wx, wy, a, b):
    a[...] = jnp.dot(x[...], wx[...], preferred_element_type=jnp.float32)
    b[...] = jnp.dot(x[...], wy[...], preferred_element_type=jnp.float32)


_prep_call = pl.pallas_call(
    _prep_body, grid=(N // BN,),
    in_specs=[pl.BlockSpec((BN, F), lambda i: (i, 0)),
              pl.BlockSpec((F, F), lambda i: (0, 0)),
              pl.BlockSpec((F, F), lambda i: (0, 0))],
    out_specs=[pl.BlockSpec((BN, F), lambda i: (i, 0)),
               pl.BlockSpec((BN, F), lambda i: (i, 0))],
    out_shape=(_f32(N, F), _f32(N, F)),
)

BE = 2000


def _edge_body(ga, gb, a8, w1e, b1r, w2, b2r, o):
    pre = (ga[...] + gb[...] + b1r[...]
           + jnp.dot(a8[...], w1e[...], preferred_element_type=jnp.float32))
    m = pre * jax.nn.sigmoid(pre)
    mm = jnp.dot(m, w2[...], preferred_element_type=jnp.float32) + b2r[...]
    o[...] = mm * jax.nn.sigmoid(mm)


_edge_call = pl.pallas_call(
    _edge_body, grid=(E // BE,),
    in_specs=[pl.BlockSpec((BE, F), lambda i: (i, 0)),
              pl.BlockSpec((BE, F), lambda i: (i, 0)),
              pl.BlockSpec((BE, 8), lambda i: (i, 0)),
              pl.BlockSpec((8, F), lambda i: (0, 0)),
              pl.BlockSpec((1, F), lambda i: (0, 0)),
              pl.BlockSpec((F, F), lambda i: (0, 0)),
              pl.BlockSpec((1, F), lambda i: (0, 0))],
    out_specs=pl.BlockSpec((BE, F), lambda i: (i, 0)),
    out_shape=_f32(E, F),
)


def _node_body(x, ap, w3x, w3a, b3r, w4, b4r, o):
    agg = ap[0] + ap[1]
    t = (jnp.dot(x[...], w3x[...], preferred_element_type=jnp.float32)
         + jnp.dot(agg, w3a[...], preferred_element_type=jnp.float32)
         + b3r[...])
    h = t * jax.nn.sigmoid(t)
    o[...] = x[...] + jnp.dot(h, w4[...], preferred_element_type=jnp.float32) + b4r[...]


_node_call = pl.pallas_call(
    _node_body, grid=(N // BN,),
    in_specs=[pl.BlockSpec((BN, F), lambda i: (i, 0)),
              pl.BlockSpec((NC, BN, F), lambda i: (0, i, 0)),
              pl.BlockSpec((F, F), lambda i: (0, 0)),
              pl.BlockSpec((F, F), lambda i: (0, 0)),
              pl.BlockSpec((1, F), lambda i: (0, 0)),
              pl.BlockSpec((F, F), lambda i: (0, 0)),
              pl.BlockSpec((1, F), lambda i: (0, 0))],
    out_specs=pl.BlockSpec((BN, F), lambda i: (i, 0)),
    out_shape=_f32(N, F),
)


def kernel(x, edge_index, edge_weight, edge_attr, W1, b1, W2, b2, W3, b3, W4, b4):
    ei = edge_index.astype(jnp.int32)
    row = ei[0]
    col = ei[1]
    a8 = jnp.concatenate(
        [edge_attr, edge_weight[:, None], jnp.zeros((E, 3), jnp.float32)], axis=1)
    w1e = jnp.concatenate(
        [W1[2 * F:2 * F + EF], W1[2 * F + EF:] * (1.0 / CUTOFF),
         jnp.zeros((3, F), jnp.float32)], axis=0)
    A, B = _prep_call(x, W1[:F], W1[F:2 * F])
    Ga, Gb = _sc_gather(A, B, row, col)
    mij = _edge_call(Ga, Gb, a8, w1e, b1[None], W2, b2[None])
    aggp = _sc_scatter(mij, row, jnp.zeros((N, F), jnp.float32))
    aggp = aggp.reshape(NC, N, F)
    return _node_call(x, aggp, W3[:F], W3[F:], b3[None], W4, b4[None])


# speedup vs baseline: 3.8608x; 1.3883x over previous
import functools

import jax
import jax.numpy as jnp
from jax import lax
from jax.experimental import pallas as pl
from jax.experimental.pallas import tpu as pltpu
from jax.experimental.pallas import tpu_sc as plsc

N, E, F, EF = 10000, 320000, 128, 4
CUTOFF = 5.0
NC, NS = 2, 16
NW = NC * NS
EPW = E // NW
C = 80
NCH = EPW // C
NPT = 624
NTAIL = N - NS * NPT

_mesh = plsc.VectorSubcoreMesh(core_axis_name="c", subcore_axis_name="s")


def _f32(*shape):
    return jax.ShapeDtypeStruct(shape, jnp.float32)


@functools.partial(
    pl.kernel, mesh=_mesh,
    out_type=(_f32(E, F), _f32(E, F)),
    scratch_types=[
        pltpu.VMEM((EPW,), jnp.int32), pltpu.VMEM((EPW,), jnp.int32),
        pltpu.VMEM((C, F), jnp.float32), pltpu.VMEM((C, F), jnp.float32),
        pltpu.VMEM((C, F), jnp.float32), pltpu.VMEM((C, F), jnp.float32),
        pltpu.SemaphoreType.DMA, pltpu.SemaphoreType.DMA,
        pltpu.SemaphoreType.DMA, pltpu.SemaphoreType.DMA,
        pltpu.SemaphoreType.DMA, pltpu.SemaphoreType.DMA,
        pltpu.SemaphoreType.DMA, pltpu.SemaphoreType.DMA,
    ],
)
def _sc_gather(ta, tb, row, col, ga, gb, rowv, colv,
               ba0, bb0, ba1, bb1,
               sga0, sgb0, sga1, sgb1, swa0, swb0, swa1, swb1):
    wid = lax.axis_index("s") * NC + lax.axis_index("c")
    base0 = wid * EPW
    pltpu.sync_copy(row.at[pl.ds(base0, EPW)], rowv)
    pltpu.sync_copy(col.at[pl.ds(base0, EPW)], colv)
    sets = ((ba0, bb0, sga0, sgb0, swa0, swb0),
            (ba1, bb1, sga1, sgb1, swa1, swb1))

    def g_start(c, s):
        ba, bb, sga, sgb = sets[s][:4]
        pltpu.async_copy(ta.at[rowv.at[pl.ds(c * C, C)]], ba, sga)
        pltpu.async_copy(tb.at[colv.at[pl.ds(c * C, C)]], bb, sgb)

    def g_wait(s):
        ba, bb, sga, sgb = sets[s][:4]
        pltpu.make_async_copy(ta.at[rowv.at[pl.ds(0, C)]], ba, sga).wait()
        pltpu.make_async_copy(tb.at[colv.at[pl.ds(0, C)]], bb, sgb).wait()

    def w_start(c, s):
        ba, bb, _, _, swa, swb = sets[s]
        b = base0 + c * C
        pltpu.async_copy(ba, ga.at[pl.ds(b, C)], swa)
        pltpu.async_copy(bb, gb.at[pl.ds(b, C)], swb)

    def w_wait(s):
        ba, bb, _, _, swa, swb = sets[s]
        pltpu.make_async_copy(ba, ga.at[pl.ds(base0, C)], swa).wait()
        pltpu.make_async_copy(bb, gb.at[pl.ds(base0, C)], swb).wait()

    g_start(0, 0)

    def body(j, carry):
        c0 = 2 * j

        @pl.when(j > 0)
        def _():
            w_wait(1)

        g_start(c0 + 1, 1)
        g_wait(0)
        w_start(c0, 0)
        w_wait(0)
        g_start(c0 + 2, 0)
        g_wait(1)
        w_start(c0 + 1, 1)
        return carry

    lax.fori_loop(0, (NCH - 1) // 2, body, 0)
    w_wait(1)
    g_wait(0)
    w_start(NCH - 1, 0)
    w_wait(0)


@functools.partial(
    pl.kernel, mesh=_mesh,
    out_type=_f32(NC * N, F),
    scratch_types=[
        pltpu.VMEM_SHARED((N, F), jnp.float32),
        pltpu.VMEM((C,), jnp.int32), pltpu.VMEM((C,), jnp.int32),
        pltpu.VMEM((C, F), jnp.float32), pltpu.VMEM((C, F), jnp.float32),
        pltpu.SemaphoreType.DMA, pltpu.SemaphoreType.DMA,
        pltpu.SemaphoreType.DMA, pltpu.SemaphoreType.DMA,
    ],
)
def _sc_scatter(mij, row, zz, aggp, acc, idx0, idx1, mb0, mb1,
                si0, sm0, si1, sm1):
    cid = lax.axis_index("c")
    sid = lax.axis_index("s")
    pltpu.sync_copy(zz.at[pl.ds(sid * NPT, NPT)], acc.at[pl.ds(sid * NPT, NPT)])

    @pl.when(sid == 0)
    def _():
        pltpu.sync_copy(zz.at[pl.ds(NS * NPT, NTAIL)],
                        acc.at[pl.ds(NS * NPT, NTAIL)])

    plsc.subcore_barrier()
    base0 = (sid * NC + cid) * EPW
    sets = ((idx0, mb0, si0, sm0), (idx1, mb1, si1, sm1))

    def l_start(c, s):
        idxv, mb, si, sm = sets[s]
        b = base0 + c * C
        pltpu.async_copy(row.at[pl.ds(b, C)], idxv, si)
        pltpu.async_copy(mij.at[pl.ds(b, C)], mb, sm)

    def l_wait(s):
        idxv, mb, si, sm = sets[s]
        pltpu.make_async_copy(row.at[pl.ds(base0, C)], idxv, si).wait()
        pltpu.make_async_copy(mij.at[pl.ds(base0, C)], mb, sm).wait()

    def scat(s):
        idxv, mb = sets[s][:2]
        pltpu.sync_copy(mb, acc.at[idxv], add=True)

    l_start(0, 0)

    def body(j, carry):
        c0 = 2 * j
        l_start(c0 + 1, 1)
        l_wait(0)
        scat(0)
        l_start(c0 + 2, 0)
        l_wait(1)
        scat(1)
        return carry

    lax.fori_loop(0, (NCH - 1) // 2, body, 0)
    l_wait(0)
    scat(0)
    plsc.subcore_barrier()
    pltpu.sync_copy(acc.at[pl.ds(sid * NPT, NPT)],
                    aggp.at[pl.ds(cid * N + sid * NPT, NPT)])

    @pl.when(sid == 0)
    def _():
        pltpu.sync_copy(acc.at[pl.ds(NS * NPT, NTAIL)],
                        aggp.at[pl.ds(cid * N + NS * NPT, NTAIL)])


BN = 1000


def _prep_body(x, wx, wy, a, b):
    a[...] = jnp.dot(x[...], wx[...], preferred_element_type=jnp.float32)
    b[...] = jnp.dot(x[...], wy[...], preferred_element_type=jnp.float32)


_prep_call = pl.pallas_call(
    _prep_body, grid=(N // BN,),
    in_specs=[pl.BlockSpec((BN, F), lambda i: (i, 0)),
              pl.BlockSpec((F, F), lambda i: (0, 0)),
              pl.BlockSpec((F, F), lambda i: (0, 0))],
    out_specs=[pl.BlockSpec((BN, F), lambda i: (i, 0)),
               pl.BlockSpec((BN, F), lambda i: (i, 0))],
    out_shape=(_f32(N, F), _f32(N, F)),
)

BE = 2000


def _edge_body(ga, gb, a8, w1e, b1r, w2, b2r, o):
    pre = (ga[...] + gb[...] + b1r[...]
           + jnp.dot(a8[...], w1e[...], preferred_element_type=jnp.float32))
    m = pre * jax.nn.sigmoid(pre)
    mm = jnp.dot(m, w2[...], preferred_element_type=jnp.float32) + b2r[...]
    o[...] = mm * jax.nn.sigmoid(mm)


_edge_call = pl.pallas_call(
    _edge_body, grid=(E // BE,),
    in_specs=[pl.BlockSpec((BE, F), lambda i: (i, 0)),
              pl.BlockSpec((BE, F), lambda i: (i, 0)),
              pl.BlockSpec((BE, 8), lambda i: (i, 0)),
              pl.BlockSpec((8, F), lambda i: (0, 0)),
              pl.BlockSpec((1, F), lambda i: (0, 0)),
              pl.BlockSpec((F, F), lambda i: (0, 0)),
              pl.BlockSpec((1, F), lambda i: (0, 0))],
    out_specs=pl.BlockSpec((BE, F), lambda i: (i, 0)),
    out_shape=_f32(E, F),
)


def _node_body(x, ap, w3x, w3a, b3r, w4, b4r, o):
    agg = ap[0] + ap[1]
    t = (jnp.dot(x[...], w3x[...], preferred_element_type=jnp.float32)
         + jnp.dot(agg, w3a[...], preferred_element_type=jnp.float32)
         + b3r[...])
    h = t * jax.nn.sigmoid(t)
    o[...] = x[...] + jnp.dot(h, w4[...], preferred_element_type=jnp.float32) + b4r[...]


_node_call = pl.pallas_call(
    _node_body, grid=(N // BN,),
    in_specs=[pl.BlockSpec((BN, F), lambda i: (i, 0)),
              pl.BlockSpec((NC, BN, F), lambda i: (0, i, 0)),
              pl.BlockSpec((F, F), lambda i: (0, 0)),
              pl.BlockSpec((F, F), lambda i: (0, 0)),
              pl.BlockSpec((1, F), lambda i: (0, 0)),
              pl.BlockSpec((F, F), lambda i: (0, 0)),
              pl.BlockSpec((1, F), lambda i: (0, 0))],
    out_specs=pl.BlockSpec((BN, F), lambda i: (i, 0)),
    out_shape=_f32(N, F),
)


def kernel(x, edge_index, edge_weight, edge_attr, W1, b1, W2, b2, W3, b3, W4, b4):
    ei = edge_index.astype(jnp.int32)
    row = ei[0]
    col = ei[1]
    a8 = jnp.concatenate(
        [edge_attr, edge_weight[:, None], jnp.zeros((E, 3), jnp.float32)], axis=1)
    w1e = jnp.concatenate(
        [W1[2 * F:2 * F + EF], W1[2 * F + EF:] * (1.0 / CUTOFF),
         jnp.zeros((3, F), jnp.float32)], axis=0)
    A, B = _prep_call(x, W1[:F], W1[F:2 * F])
    Ga, Gb = _sc_gather(A, B, row, col)
    mij = _edge_call(Ga, Gb, a8, w1e, b1[None], W2, b2[None])
    aggp = _sc_scatter(mij, row, jnp.zeros((N, F), jnp.float32))
    aggp = aggp.reshape(NC, N, F)
    return _node_call(x, aggp, W3[:F], W3[F:], b3[None], W4, b4[None])


# speedup vs baseline: 4.8477x; 1.2556x over previous
import functools

import jax
import jax.numpy as jnp
from jax import lax
from jax.experimental import pallas as pl
from jax.experimental.pallas import tpu as pltpu
from jax.experimental.pallas import tpu_sc as plsc

N, E, F, EF = 10000, 320000, 128, 4
CUTOFF = 5.0
NC, NS = 2, 16
NW = NC * NS
C = 80
NPT = 624
NTAIL = N - NS * NPT

NCH1, NCH2 = 63, 62
E1, E2 = NW * C * NCH1, NW * C * NCH2
assert E1 + E2 == E

_mesh = plsc.VectorSubcoreMesh(core_axis_name="c", subcore_axis_name="s")


def _f32(*shape):
    return jax.ShapeDtypeStruct(shape, jnp.float32)


def _make_gather(nch):
    eseg = NW * C * nch
    epw = C * nch

    @functools.partial(
        pl.kernel, mesh=_mesh,
        out_type=_f32(eseg, F),
        scratch_types=[
            pltpu.VMEM((epw,), jnp.int32), pltpu.VMEM((epw,), jnp.int32),
            pltpu.VMEM((C, F), jnp.float32), pltpu.VMEM((C, F), jnp.float32),
            pltpu.VMEM((C, F), jnp.float32), pltpu.VMEM((C, F), jnp.float32),
            pltpu.SemaphoreType.DMA, pltpu.SemaphoreType.DMA,
            pltpu.SemaphoreType.DMA, pltpu.SemaphoreType.DMA,
            pltpu.SemaphoreType.DMA, pltpu.SemaphoreType.DMA,
        ],
    )
    def k(ta, tb, row, col, gs, rowv, colv,
          ba0, bb0, ba1, bb1,
          sga0, sgb0, sga1, sgb1, sw0, sw1):
        wid = lax.axis_index("s") * NC + lax.axis_index("c")
        base0 = wid * epw
        pltpu.sync_copy(row.at[pl.ds(base0, epw)], rowv)
        pltpu.sync_copy(col.at[pl.ds(base0, epw)], colv)
        sets = ((ba0, bb0, sga0, sgb0, sw0),
                (ba1, bb1, sga1, sgb1, sw1))

        def g_start(c, s):
            ba, bb, sga, sgb = sets[s][:4]
            pltpu.async_copy(ta.at[rowv.at[pl.ds(c * C, C)]], ba, sga)
            pltpu.async_copy(tb.at[colv.at[pl.ds(c * C, C)]], bb, sgb)

        def g_wait(s):
            ba, bb, sga, sgb = sets[s][:4]
            pltpu.make_async_copy(ta.at[rowv.at[pl.ds(0, C)]], ba, sga).wait()
            pltpu.make_async_copy(tb.at[colv.at[pl.ds(0, C)]], bb, sgb).wait()

        def add_bufs(s):
            ba, bb = sets[s][:2]

            def rbody(r, carry):
                for u in range(4):
                    rr = r * 4 + u
                    for k8 in range(F // 16):
                        sl = pl.ds(k8 * 16, 16)
                        ba[rr, sl] = ba[rr, sl] + bb[rr, sl]
                return carry

            lax.fori_loop(0, C // 4, rbody, 0)

        def w_start(c, s):
            ba, sw = sets[s][0], sets[s][4]
            pltpu.async_copy(ba, gs.at[pl.ds(base0 + c * C, C)], sw)

        def w_wait(s):
            ba, sw = sets[s][0], sets[s][4]
            pltpu.make_async_copy(ba, gs.at[pl.ds(base0, C)], sw).wait()

        g_start(0, 0)

        def body(j, carry):
            c0 = 2 * j

            @pl.when(j > 0)
            def _():
                w_wait(1)

            g_start(c0 + 1, 1)
            g_wait(0)
            add_bufs(0)
            w_start(c0, 0)
            w_wait(0)
            g_start(c0 + 2, 0)
            g_wait(1)
            add_bufs(1)
            w_start(c0 + 1, 1)
            return carry

        nfull = (nch - 1) // 2
        lax.fori_loop(0, nfull, body, 0)
        w_wait(1)
        if nch % 2 == 1:
            g_wait(0)
            add_bufs(0)
            w_start(nch - 1, 0)
            w_wait(0)
        else:
            g_start(nch - 1, 1)
            g_wait(0)
            add_bufs(0)
            w_start(nch - 2, 0)
            g_wait(1)
            add_bufs(1)
            w_start(nch - 1, 1)
            w_wait(0)
            w_wait(1)

    return k


_gather1 = _make_gather(NCH1)
_gather2 = _make_gather(NCH2)


def _make_scatter(nch):
    eseg = NW * C * nch
    epw = C * nch

    @functools.partial(
        pl.kernel, mesh=_mesh,
        out_type=_f32(NC * N, F),
        scratch_types=[
            pltpu.VMEM_SHARED((N, F), jnp.float32),
            pltpu.VMEM((C,), jnp.int32), pltpu.VMEM((C,), jnp.int32),
            pltpu.VMEM((C, F), jnp.float32), pltpu.VMEM((C, F), jnp.float32),
            pltpu.SemaphoreType.DMA, pltpu.SemaphoreType.DMA,
            pltpu.SemaphoreType.DMA, pltpu.SemaphoreType.DMA,
        ],
    )
    def k(mij, row, zz, aggp, acc, idx0, idx1, mb0, mb1, si0, sm0, si1, sm1):
        cid = lax.axis_index("c")
        sid = lax.axis_index("s")
        pltpu.sync_copy(zz.at[pl.ds(sid * NPT, NPT)],
                        acc.at[pl.ds(sid * NPT, NPT)])

        @pl.when(sid == 0)
        def _():
            pltpu.sync_copy(zz.at[pl.ds(NS * NPT, NTAIL)],
                            acc.at[pl.ds(NS * NPT, NTAIL)])

        plsc.subcore_barrier()
        base0 = (sid * NC + cid) * epw
        sets = ((idx0, mb0, si0, sm0), (idx1, mb1, si1, sm1))

        def l_start(c, s):
            idxv, mb, si, sm = sets[s]
            b = base0 + c * C
            pltpu.async_copy(row.at[pl.ds(b, C)], idxv, si)
            pltpu.async_copy(mij.at[pl.ds(b, C)], mb, sm)

        def l_wait(s):
            idxv, mb, si, sm = sets[s]
            pltpu.make_async_copy(row.at[pl.ds(base0, C)], idxv, si).wait()
            pltpu.make_async_copy(mij.at[pl.ds(base0, C)], mb, sm).wait()

        def scat(s):
            idxv, mb = sets[s][:2]
            pltpu.sync_copy(mb, acc.at[idxv], add=True)

        l_start(0, 0)

        def body(j, carry):
            c0 = 2 * j
            l_start(c0 + 1, 1)
            l_wait(0)
            scat(0)
            l_start(c0 + 2, 0)
            l_wait(1)
            scat(1)
            return carry

        nfull = (nch - 1) // 2
        lax.fori_loop(0, nfull, body, 0)
        l_wait(0)
        scat(0)
        if nch % 2 == 0:
            l_start(nch - 1, 1)
            l_wait(1)
            scat(1)
        plsc.subcore_barrier()
        pltpu.sync_copy(acc.at[pl.ds(sid * NPT, NPT)],
                        aggp.at[pl.ds(cid * N + sid * NPT, NPT)])

        @pl.when(sid == 0)
        def _():
            pltpu.sync_copy(acc.at[pl.ds(NS * NPT, NTAIL)],
                            aggp.at[pl.ds(cid * N + NS * NPT, NTAIL)])

    return k


_scatter1 = _make_scatter(NCH1)
_scatter2 = _make_scatter(NCH2)


BN = 1000


def _prep_body(x, wx, wy, a, b):
    a[...] = jnp.dot(x[...], wx[...], preferred_element_type=jnp.float32)
    b[...] = jnp.dot(x[...], wy[...], preferred_element_type=jnp.float32)


_prep_call = pl.pallas_call(
    _prep_body, grid=(N // BN,),
    in_specs=[pl.BlockSpec((BN, F), lambda i: (i, 0)),
              pl.BlockSpec((F, F), lambda i: (0, 0)),
              pl.BlockSpec((F, F), lambda i: (0, 0))],
    out_specs=[pl.BlockSpec((BN, F), lambda i: (i, 0)),
               pl.BlockSpec((BN, F), lambda i: (i, 0))],
    out_shape=(_f32(N, F), _f32(N, F)),
)

BE = NW * C


def _edge_body(g, a8, w1e, b1r, w2, b2r, o):
    pre = (g[...] + b1r[...]
           + jnp.dot(a8[...], w1e[...], preferred_element_type=jnp.float32))
    m = pre * jax.nn.sigmoid(pre)
    mm = jnp.dot(m, w2[...], preferred_element_type=jnp.float32) + b2r[...]
    o[...] = mm * jax.nn.sigmoid(mm)


def _make_edge(nch):
    eseg = NW * C * nch
    return pl.pallas_call(
        _edge_body, grid=(nch,),
        in_specs=[pl.BlockSpec((BE, F), lambda i: (i, 0)),
                  pl.BlockSpec((BE, 8), lambda i: (i, 0)),
                  pl.BlockSpec((8, F), lambda i: (0, 0)),
                  pl.BlockSpec((1, F), lambda i: (0, 0)),
                  pl.BlockSpec((F, F), lambda i: (0, 0)),
                  pl.BlockSpec((1, F), lambda i: (0, 0))],
        out_specs=pl.BlockSpec((BE, F), lambda i: (i, 0)),
        out_shape=_f32(eseg, F),
    )


_edge1 = _make_edge(NCH1)
_edge2 = _make_edge(NCH2)


def _node_body(x, ap1, ap2, w3x, w3a, b3r, w4, b4r, o):
    agg = ap1[0] + ap1[1] + ap2[0] + ap2[1]
    t = (jnp.dot(x[...], w3x[...], preferred_element_type=jnp.float32)
         + jnp.dot(agg, w3a[...], preferred_element_type=jnp.float32)
         + b3r[...])
    h = t * jax.nn.sigmoid(t)
    o[...] = x[...] + jnp.dot(h, w4[...], preferred_element_type=jnp.float32) + b4r[...]


_node_call = pl.pallas_call(
    _node_body, grid=(N // BN,),
    in_specs=[pl.BlockSpec((BN, F), lambda i: (i, 0)),
              pl.BlockSpec((NC, BN, F), lambda i: (0, i, 0)),
              pl.BlockSpec((NC, BN, F), lambda i: (0, i, 0)),
              pl.BlockSpec((F, F), lambda i: (0, 0)),
              pl.BlockSpec((F, F), lambda i: (0, 0)),
              pl.BlockSpec((1, F), lambda i: (0, 0)),
              pl.BlockSpec((F, F), lambda i: (0, 0)),
              pl.BlockSpec((1, F), lambda i: (0, 0))],
    out_specs=pl.BlockSpec((BN, F), lambda i: (i, 0)),
    out_shape=_f32(N, F),
)


def kernel(x, edge_index, edge_weight, edge_attr, W1, b1, W2, b2, W3, b3, W4, b4):
    ei = edge_index.astype(jnp.int32)
    row = ei[0]
    col = ei[1]
    a8 = jnp.concatenate(
        [edge_attr, edge_weight[:, None], jnp.zeros((E, 3), jnp.float32)], axis=1)
    w1e = jnp.concatenate(
        [W1[2 * F:2 * F + EF], W1[2 * F + EF:] * (1.0 / CUTOFF),
         jnp.zeros((3, F), jnp.float32)], axis=0)
    A, B = _prep_call(x, W1[:F], W1[F:2 * F])
    zz = jnp.zeros((N, F), jnp.float32)
    b1r, b2r = b1[None], b2[None]

    G1 = _gather1(A, B, row[:E1], col[:E1])
    G2 = _gather2(A, B, row[E1:], col[E1:])
    mij1 = _edge1(G1, a8[:E1], w1e, b1r, W2, b2r)
    mij2 = _edge2(G2, a8[E1:], w1e, b1r, W2, b2r)
    aggp1 = _scatter1(mij1, row[:E1], zz).reshape(NC, N, F)
    aggp2 = _scatter2(mij2, row[E1:], zz).reshape(NC, N, F)
    return _node_call(x, aggp1, aggp2, W3[:F], W3[F:], b3[None], W4, b4[None])


# speedup vs baseline: 4.8872x; 1.0082x over previous
import functools

import jax
import jax.numpy as jnp
from jax import lax
from jax.experimental import pallas as pl
from jax.experimental.pallas import tpu as pltpu
from jax.experimental.pallas import tpu_sc as plsc

N, E, F, EF = 10000, 320000, 128, 4
CUTOFF = 5.0
NC, NS = 2, 16
NW = NC * NS
C = 80
NPT = 624
NTAIL = N - NS * NPT

NCH1, NCH2 = 63, 62
E1, E2 = NW * C * NCH1, NW * C * NCH2
assert E1 + E2 == E

_mesh = plsc.VectorSubcoreMesh(core_axis_name="c", subcore_axis_name="s")


def _f32(*shape):
    return jax.ShapeDtypeStruct(shape, jnp.float32)


def _make_gather(nch):
    eseg = NW * C * nch
    epw = C * nch

    @functools.partial(
        pl.kernel, mesh=_mesh,
        out_type=_f32(eseg, F),
        scratch_types=[
            pltpu.VMEM((epw,), jnp.int32), pltpu.VMEM((epw,), jnp.int32),
            pltpu.VMEM((C, F), jnp.float32), pltpu.VMEM((C, F), jnp.float32),
            pltpu.VMEM((C, F), jnp.float32), pltpu.VMEM((C, F), jnp.float32),
            pltpu.SemaphoreType.DMA, pltpu.SemaphoreType.DMA,
            pltpu.SemaphoreType.DMA, pltpu.SemaphoreType.DMA,
            pltpu.SemaphoreType.DMA, pltpu.SemaphoreType.DMA,
        ],
    )
    def k(ta, tb, row, col, gs, rowv, colv,
          ba0, bb0, ba1, bb1,
          sga0, sgb0, sga1, sgb1, sw0, sw1):
        wid = lax.axis_index("s") * NC + lax.axis_index("c")
        base0 = wid * epw
        pltpu.sync_copy(row.at[pl.ds(base0, epw)], rowv)
        pltpu.sync_copy(col.at[pl.ds(base0, epw)], colv)
        sets = ((ba0, bb0, sga0, sgb0, sw0),
                (ba1, bb1, sga1, sgb1, sw1))

        def g_start(c, s):
            ba, bb, sga, sgb = sets[s][:4]
            pltpu.async_copy(ta.at[rowv.at[pl.ds(c * C, C)]], ba, sga)
            pltpu.async_copy(tb.at[colv.at[pl.ds(c * C, C)]], bb, sgb)

        def g_wait(s):
            ba, bb, sga, sgb = sets[s][:4]
            pltpu.make_async_copy(ta.at[rowv.at[pl.ds(0, C)]], ba, sga).wait()
            pltpu.make_async_copy(tb.at[colv.at[pl.ds(0, C)]], bb, sgb).wait()

        def add_bufs(s):
            ba, bb = sets[s][:2]

            def rbody(r, carry):
                for u in range(4):
                    rr = r * 4 + u
                    for k8 in range(F // 16):
                        sl = pl.ds(k8 * 16, 16)
                        ba[rr, sl] = ba[rr, sl] + bb[rr, sl]
                return carry

            lax.fori_loop(0, C // 4, rbody, 0)

        def w_start(c, s):
            ba, sw = sets[s][0], sets[s][4]
            pltpu.async_copy(ba, gs.at[pl.ds(base0 + c * C, C)], sw)

        def w_wait(s):
            ba, sw = sets[s][0], sets[s][4]
            pltpu.make_async_copy(ba, gs.at[pl.ds(base0, C)], sw).wait()

        g_start(0, 0)

        def body(j, carry):
            c0 = 2 * j

            @pl.when(j > 0)
            def _():
                w_wait(1)

            g_start(c0 + 1, 1)
            g_wait(0)
            add_bufs(0)
            w_start(c0, 0)
            w_wait(0)
            g_start(c0 + 2, 0)
            g_wait(1)
            add_bufs(1)
            w_start(c0 + 1, 1)
            return carry

        nfull = (nch - 1) // 2
        lax.fori_loop(0, nfull, body, 0)
        w_wait(1)
        if nch % 2 == 1:
            g_wait(0)
            add_bufs(0)
            w_start(nch - 1, 0)
            w_wait(0)
        else:
            g_start(nch - 1, 1)
            g_wait(0)
            add_bufs(0)
            w_start(nch - 2, 0)
            g_wait(1)
            add_bufs(1)
            w_start(nch - 1, 1)
            w_wait(0)
            w_wait(1)

    return k


_gather1 = _make_gather(NCH1)
_gather2 = _make_gather(NCH2)


def _make_scatter(nch):
    eseg = NW * C * nch
    epw = C * nch

    @functools.partial(
        pl.kernel, mesh=_mesh,
        out_type=_f32(NC * N, F),
        scratch_types=[
            pltpu.VMEM_SHARED((N, F), jnp.float32),
            pltpu.VMEM((C,), jnp.int32), pltpu.VMEM((C,), jnp.int32),
            pltpu.VMEM((C, F), jnp.float32), pltpu.VMEM((C, F), jnp.float32),
            pltpu.SemaphoreType.DMA, pltpu.SemaphoreType.DMA,
            pltpu.SemaphoreType.DMA, pltpu.SemaphoreType.DMA,
        ],
    )
    def k(mij, row, init, aggp, acc, idx0, idx1, mb0, mb1, si0, sm0, si1, sm1):
        cid = lax.axis_index("c")
        sid = lax.axis_index("s")
        pltpu.sync_copy(init.at[pl.ds(cid * N + sid * NPT, NPT)],
                        acc.at[pl.ds(sid * NPT, NPT)])

        @pl.when(sid == 0)
        def _():
            pltpu.sync_copy(init.at[pl.ds(cid * N + NS * NPT, NTAIL)],
                            acc.at[pl.ds(NS * NPT, NTAIL)])

        plsc.subcore_barrier()
        base0 = (sid * NC + cid) * epw
        sets = ((idx0, mb0, si0, sm0), (idx1, mb1, si1, sm1))

        def l_start(c, s):
            idxv, mb, si, sm = sets[s]
            b = base0 + c * C
            pltpu.async_copy(row.at[pl.ds(b, C)], idxv, si)
            pltpu.async_copy(mij.at[pl.ds(b, C)], mb, sm)

        def l_wait(s):
            idxv, mb, si, sm = sets[s]
            pltpu.make_async_copy(row.at[pl.ds(base0, C)], idxv, si).wait()
            pltpu.make_async_copy(mij.at[pl.ds(base0, C)], mb, sm).wait()

        def scat(s):
            idxv, mb = sets[s][:2]
            pltpu.sync_copy(mb, acc.at[idxv], add=True)

        l_start(0, 0)

        def body(j, carry):
            c0 = 2 * j
            l_start(c0 + 1, 1)
            l_wait(0)
            scat(0)
            l_start(c0 + 2, 0)
            l_wait(1)
            scat(1)
            return carry

        nfull = (nch - 1) // 2
        lax.fori_loop(0, nfull, body, 0)
        l_wait(0)
        scat(0)
        if nch % 2 == 0:
            l_start(nch - 1, 1)
            l_wait(1)
            scat(1)
        plsc.subcore_barrier()
        pltpu.sync_copy(acc.at[pl.ds(sid * NPT, NPT)],
                        aggp.at[pl.ds(cid * N + sid * NPT, NPT)])

        @pl.when(sid == 0)
        def _():
            pltpu.sync_copy(acc.at[pl.ds(NS * NPT, NTAIL)],
                            aggp.at[pl.ds(cid * N + NS * NPT, NTAIL)])

    return k


_scatter1 = _make_scatter(NCH1)
_scatter2 = _make_scatter(NCH2)


BN = 1000


def _prep_body(x, wx, wy, a, b):
    a[...] = jnp.dot(x[...], wx[...], preferred_element_type=jnp.float32)
    b[...] = jnp.dot(x[...], wy[...], preferred_element_type=jnp.float32)


_prep_call = pl.pallas_call(
    _prep_body, grid=(N // BN,),
    in_specs=[pl.BlockSpec((BN, F), lambda i: (i, 0)),
              pl.BlockSpec((F, F), lambda i: (0, 0)),
              pl.BlockSpec((F, F), lambda i: (0, 0))],
    out_specs=[pl.BlockSpec((BN, F), lambda i: (i, 0)),
               pl.BlockSpec((BN, F), lambda i: (i, 0))],
    out_shape=(_f32(N, F), _f32(N, F)),
)

BE = NW * C


def _edge_body(g, a8, w1e, b1r, w2, b2r, o):
    pre = (g[...] + b1r[...]
           + jnp.dot(a8[...], w1e[...], preferred_element_type=jnp.float32))
    m = pre * jax.nn.sigmoid(pre)
    mm = jnp.dot(m, w2[...], preferred_element_type=jnp.float32) + b2r[...]
    o[...] = mm * jax.nn.sigmoid(mm)


def _make_edge(nch):
    eseg = NW * C * nch
    return pl.pallas_call(
        _edge_body, grid=(nch,),
        in_specs=[pl.BlockSpec((BE, F), lambda i: (i, 0)),
                  pl.BlockSpec((BE, 8), lambda i: (i, 0)),
                  pl.BlockSpec((8, F), lambda i: (0, 0)),
                  pl.BlockSpec((1, F), lambda i: (0, 0)),
                  pl.BlockSpec((F, F), lambda i: (0, 0)),
                  pl.BlockSpec((1, F), lambda i: (0, 0))],
        out_specs=pl.BlockSpec((BE, F), lambda i: (i, 0)),
        out_shape=_f32(eseg, F),
    )


_edge1 = _make_edge(NCH1)
_edge2 = _make_edge(NCH2)


def _node_body(x, ap, w3x, w3a, b3r, w4, b4r, o):
    agg = ap[0] + ap[1]
    t = (jnp.dot(x[...], w3x[...], preferred_element_type=jnp.float32)
         + jnp.dot(agg, w3a[...], preferred_element_type=jnp.float32)
         + b3r[...])
    h = t * jax.nn.sigmoid(t)
    o[...] = x[...] + jnp.dot(h, w4[...], preferred_element_type=jnp.float32) + b4r[...]


_node_call = pl.pallas_call(
    _node_body, grid=(N // BN,),
    in_specs=[pl.BlockSpec((BN, F), lambda i: (i, 0)),
              pl.BlockSpec((NC, BN, F), lambda i: (0, i, 0)),
              pl.BlockSpec((F, F), lambda i: (0, 0)),
              pl.BlockSpec((F, F), lambda i: (0, 0)),
              pl.BlockSpec((1, F), lambda i: (0, 0)),
              pl.BlockSpec((F, F), lambda i: (0, 0)),
              pl.BlockSpec((1, F), lambda i: (0, 0))],
    out_specs=pl.BlockSpec((BN, F), lambda i: (i, 0)),
    out_shape=_f32(N, F),
)


def kernel(x, edge_index, edge_weight, edge_attr, W1, b1, W2, b2, W3, b3, W4, b4):
    ei = edge_index.astype(jnp.int32)
    row = ei[0]
    col = ei[1]
    a8 = jnp.concatenate(
        [edge_attr, edge_weight[:, None], jnp.zeros((E, 3), jnp.float32)], axis=1)
    w1e = jnp.concatenate(
        [W1[2 * F:2 * F + EF], W1[2 * F + EF:] * (1.0 / CUTOFF),
         jnp.zeros((3, F), jnp.float32)], axis=0)
    A, B = _prep_call(x, W1[:F], W1[F:2 * F])
    zz = jnp.zeros((NC * N, F), jnp.float32)
    b1r, b2r = b1[None], b2[None]

    G1 = _gather1(A, B, row[:E1], col[:E1])
    G2 = _gather2(A, B, row[E1:], col[E1:])
    mij1 = _edge1(G1, a8[:E1], w1e, b1r, W2, b2r)
    mij2 = _edge2(G2, a8[E1:], w1e, b1r, W2, b2r)
    aggp1 = _scatter1(mij1, row[:E1], zz)
    aggp2 = _scatter2(mij2, row[E1:], aggp1).reshape(NC, N, F)
    return _node_call(x, aggp2, W3[:F], W3[F:], b3[None], W4, b4[None])
